# Initial kernel scaffold; baseline (speedup 1.0000x reference)
import jax
import jax.numpy as jnp
from jax.experimental import pallas as pl


def kernel(hidden_states_c, hidden_states_h, instruction_pointer, attribution, current_step, node_embeddings, docstring_embeddings, docstring_mask, edge_sources, edge_dests, edge_types, true_indexes, false_indexes, raise_indexes, exit_node_indexes, raise_node_indexes, step_limits, Wx, Wh, b_lstm, W_raise, b_raise, W_branch, b_branch):
    raise NotImplementedError("write your pallas kernel here")


# speedup vs baseline: 1.0542x; 1.0542x over previous
import functools

import jax
import jax.numpy as jnp
from jax.experimental import pallas as pl
from jax.experimental.pallas import tpu as pltpu

B, N, H = 8, 16384, 64
ROWS = B * N
BLK = 1024


def _lstm_body(emb_ref, h_ref, c_ref, ip_ref, wx_ref, wh_ref, bl_ref,
               wrc_ref, wrh_ref, wbc_ref, wbh_ref, dbr_ref, dbb_ref,
               c2_ref, h2_ref, wr_ref, wt_ref, wf_ref):
    emb = emb_ref[...]
    h = h_ref[...]
    c = c_ref[...]
    z = (jnp.dot(emb, wx_ref[...], preferred_element_type=jnp.float32)
         + jnp.dot(h, wh_ref[...], preferred_element_type=jnp.float32)
         + bl_ref[...])
    i = jax.nn.sigmoid(z[:, 0:H])
    f = jax.nn.sigmoid(z[:, H:2 * H])
    g = jnp.tanh(z[:, 2 * H:3 * H])
    o = jax.nn.sigmoid(z[:, 3 * H:4 * H])
    c2 = f * c + i * g
    h2 = o * jnp.tanh(c2)
    c2_ref[...] = c2
    h2_ref[...] = h2
    rd = jnp.sum(c2 * wrc_ref[...] + h2 * wrh_ref[...], axis=1) + dbr_ref[0]
    bd = jnp.sum(c2 * wbc_ref[...] + h2 * wbh_ref[...], axis=1) + dbb_ref[0]
    p_raise = jax.nn.sigmoid(rd)
    p_true = jax.nn.sigmoid(bd)
    ip = ip_ref[...]
    w_r = p_raise * ip
    p_nr_ip = ip - w_r
    w_t = p_nr_ip * p_true
    wr_ref[...] = w_r
    wt_ref[...] = w_t
    wf_ref[...] = p_nr_ip - w_t


def _lstm_phase(emb, h, c, ip, Wx, Wh, b_lstm, W_raise, b_raise, W_branch, b_branch):
    wrc = (W_raise[0:H, 0] - W_raise[0:H, 1]).reshape(1, H)
    wrh = (W_raise[H:2 * H, 0] - W_raise[H:2 * H, 1]).reshape(1, H)
    wbc = (W_branch[0:H, 0] - W_branch[0:H, 1]).reshape(1, H)
    wbh = (W_branch[H:2 * H, 0] - W_branch[H:2 * H, 1]).reshape(1, H)
    dbr = (b_raise[0] - b_raise[1]).reshape(1)
    dbb = (b_branch[0] - b_branch[1]).reshape(1)

    grid = (ROWS // BLK,)
    row_spec = pl.BlockSpec((BLK, H), lambda i: (i, 0))
    vec_spec = pl.BlockSpec((BLK,), lambda i: (i,))
    full2 = lambda a: pl.BlockSpec(a.shape, lambda i: tuple(0 for _ in a.shape))
    out = pl.pallas_call(
        _lstm_body,
        grid=grid,
        in_specs=[row_spec, row_spec, row_spec, vec_spec,
                  full2(Wx), full2(Wh), pl.BlockSpec((1, 4 * H), lambda i: (0, 0)),
                  full2(wrc), full2(wrh), full2(wbc), full2(wbh),
                  full2(dbr), full2(dbb)],
        out_specs=[row_spec, row_spec, vec_spec, vec_spec, vec_spec],
        out_shape=[jax.ShapeDtypeStruct((ROWS, H), jnp.float32),
                   jax.ShapeDtypeStruct((ROWS, H), jnp.float32),
                   jax.ShapeDtypeStruct((ROWS,), jnp.float32),
                   jax.ShapeDtypeStruct((ROWS,), jnp.float32),
                   jax.ShapeDtypeStruct((ROWS,), jnp.float32)],
    )(emb.reshape(ROWS, H), h.reshape(ROWS, H), c.reshape(ROWS, H),
      ip.reshape(ROWS), Wx, Wh, b_lstm.reshape(1, 4 * H),
      wrc, wrh, wbc, wbh, dbr, dbb)
    return out


def kernel(hidden_states_c, hidden_states_h, instruction_pointer, attribution,
           current_step, node_embeddings, docstring_embeddings, docstring_mask,
           edge_sources, edge_dests, edge_types, true_indexes, false_indexes,
           raise_indexes, exit_node_indexes, raise_node_indexes, step_limits,
           Wx, Wh, b_lstm, W_raise, b_raise, W_branch, b_branch):
    c2, h2, w_r, w_t, w_f = _lstm_phase(
        node_embeddings, hidden_states_h, hidden_states_c, instruction_pointer,
        Wx, Wh, b_lstm, W_raise, b_raise, W_branch, b_branch)
    c2 = c2.reshape(B, N, H)
    h2 = h2.reshape(B, N, H)
    w_r = w_r.reshape(B, N)
    w_t = w_t.reshape(B, N)
    w_f = w_f.reshape(B, N)

    def per_example(wr, wt, wf, ti, fi, ri, c_new, h_new, c_old, h_old, exit_idx, raise_idx):
        rc = jax.ops.segment_sum(wr, ri, num_segments=N)
        tc = jax.ops.segment_sum(wt, ti, num_segments=N)
        fc = jax.ops.segment_sum(wf, fi, num_segments=N)
        ip_new = rc + tc + fc
        denom = ip_new + 1e-07

        def agg(x):
            r = jax.ops.segment_sum(x * wr[:, None], ri, num_segments=N)
            t = jax.ops.segment_sum(x * wt[:, None], ti, num_segments=N)
            f = jax.ops.segment_sum(x * wf[:, None], fi, num_segments=N)
            return (r + t + f) / denom[:, None]
        c_agg = agg(c_new)
        h_agg = agg(h_new)
        c_agg = c_agg.at[exit_idx, :].set(c_old[exit_idx, :]).at[raise_idx, :].set(c_old[raise_idx, :])
        h_agg = h_agg.at[exit_idx, :].set(h_old[exit_idx, :]).at[raise_idx, :].set(h_old[raise_idx, :])
        return ip_new, c_agg, h_agg

    ip_new, c_out, h_out = jax.vmap(per_example)(
        w_r, w_t, w_f, true_indexes, false_indexes, raise_indexes,
        c2, h2, hidden_states_c, hidden_states_h,
        exit_node_indexes, raise_node_indexes)
    return c_out, h_out, ip_new, attribution, current_step + 1


# speedup vs baseline: 1.0558x; 1.0016x over previous
import jax
import jax.numpy as jnp
from jax.experimental import pallas as pl

B, N, H = 8, 16384, 64
ROWS = B * N
BLK = 1024


def _lstm_body(emb_ref, h_ref, c_ref, ip_ref, wx_ref, wh_ref, bl_ref,
               wrc_ref, wrh_ref, wbc_ref, wbh_ref, dbr_ref, dbb_ref,
               c2_ref, h2_ref, wr_ref, wt_ref, wf_ref):
    emb = emb_ref[...]
    h = h_ref[...]
    c = c_ref[...]
    z = (jnp.dot(emb, wx_ref[...], preferred_element_type=jnp.float32)
         + jnp.dot(h, wh_ref[...], preferred_element_type=jnp.float32)
         + bl_ref[...])
    i = jax.nn.sigmoid(z[:, 0:H])
    f = jax.nn.sigmoid(z[:, H:2 * H])
    g = jnp.tanh(z[:, 2 * H:3 * H])
    o = jax.nn.sigmoid(z[:, 3 * H:4 * H])
    c2 = f * c + i * g
    h2 = o * jnp.tanh(c2)
    c2_ref[...] = c2
    h2_ref[...] = h2
    rd = jnp.sum(c2 * wrc_ref[...] + h2 * wrh_ref[...], axis=1) + dbr_ref[0]
    bd = jnp.sum(c2 * wbc_ref[...] + h2 * wbh_ref[...], axis=1) + dbb_ref[0]
    p_raise = jax.nn.sigmoid(rd)
    p_true = jax.nn.sigmoid(bd)
    ip = ip_ref[...]
    w_r = p_raise * ip
    p_nr_ip = ip - w_r
    w_t = p_nr_ip * p_true
    wr_ref[...] = w_r
    wt_ref[...] = w_t
    wf_ref[...] = p_nr_ip - w_t


def _lstm_phase(emb, h, c, ip, Wx, Wh, b_lstm, W_raise, b_raise, W_branch, b_branch):
    wrc = (W_raise[0:H, 0] - W_raise[0:H, 1]).reshape(1, H)
    wrh = (W_raise[H:2 * H, 0] - W_raise[H:2 * H, 1]).reshape(1, H)
    wbc = (W_branch[0:H, 0] - W_branch[0:H, 1]).reshape(1, H)
    wbh = (W_branch[H:2 * H, 0] - W_branch[H:2 * H, 1]).reshape(1, H)
    dbr = (b_raise[0] - b_raise[1]).reshape(1)
    dbb = (b_branch[0] - b_branch[1]).reshape(1)

    grid = (ROWS // BLK,)
    row_spec = pl.BlockSpec((BLK, H), lambda i: (i, 0))
    vec_spec = pl.BlockSpec((BLK,), lambda i: (i,))
    full2 = lambda a: pl.BlockSpec(a.shape, lambda i: tuple(0 for _ in a.shape))
    out = pl.pallas_call(
        _lstm_body,
        grid=grid,
        in_specs=[row_spec, row_spec, row_spec, vec_spec,
                  full2(Wx), full2(Wh), pl.BlockSpec((1, 4 * H), lambda i: (0, 0)),
                  full2(wrc), full2(wrh), full2(wbc), full2(wbh),
                  full2(dbr), full2(dbb)],
        out_specs=[row_spec, row_spec, vec_spec, vec_spec, vec_spec],
        out_shape=[jax.ShapeDtypeStruct((ROWS, H), jnp.float32),
                   jax.ShapeDtypeStruct((ROWS, H), jnp.float32),
                   jax.ShapeDtypeStruct((ROWS,), jnp.float32),
                   jax.ShapeDtypeStruct((ROWS,), jnp.float32),
                   jax.ShapeDtypeStruct((ROWS,), jnp.float32)],
    )(emb.reshape(ROWS, H), h.reshape(ROWS, H), c.reshape(ROWS, H),
      ip.reshape(ROWS), Wx, Wh, b_lstm.reshape(1, 4 * H),
      wrc, wrh, wbc, wbh, dbr, dbb)
    return out


def kernel(hidden_states_c, hidden_states_h, instruction_pointer, attribution,
           current_step, node_embeddings, docstring_embeddings, docstring_mask,
           edge_sources, edge_dests, edge_types, true_indexes, false_indexes,
           raise_indexes, exit_node_indexes, raise_node_indexes, step_limits,
           Wx, Wh, b_lstm, W_raise, b_raise, W_branch, b_branch):
    c2, h2, w_r, w_t, w_f = _lstm_phase(
        node_embeddings, hidden_states_h, hidden_states_c, instruction_pointer,
        Wx, Wh, b_lstm, W_raise, b_raise, W_branch, b_branch)
    c2 = c2.reshape(B, N, H)
    h2 = h2.reshape(B, N, H)
    w_r = w_r.reshape(B, N)
    w_t = w_t.reshape(B, N)
    w_f = w_f.reshape(B, N)

    def per_example(wr, wt, wf, ti, fi, ri, c_new, h_new, c_old, h_old, exit_idx, raise_idx):
        rc = jax.ops.segment_sum(wr, ri, num_segments=N)
        tc = jax.ops.segment_sum(wt, ti, num_segments=N)
        fc = jax.ops.segment_sum(wf, fi, num_segments=N)
        ip_new = rc + tc + fc
        denom = ip_new + 1e-07

        def agg(x):
            r = jax.ops.segment_sum(x * wr[:, None], ri, num_segments=N)
            t = jax.ops.segment_sum(x * wt[:, None], ti, num_segments=N)
            f = jax.ops.segment_sum(x * wf[:, None], fi, num_segments=N)
            return (r + t + f) / denom[:, None]
        c_agg = agg(c_new)
        h_agg = agg(h_new)
        c_agg = c_agg.at[exit_idx, :].set(c_old[exit_idx, :]).at[raise_idx, :].set(c_old[raise_idx, :])
        h_agg = h_agg.at[exit_idx, :].set(h_old[exit_idx, :]).at[raise_idx, :].set(h_old[raise_idx, :])
        return ip_new, c_agg, h_agg

    ip_new, c_out, h_out = jax.vmap(per_example)(
        w_r, w_t, w_f, true_indexes, false_indexes, raise_indexes,
        c2, h2, hidden_states_c, hidden_states_h,
        exit_node_indexes, raise_node_indexes)
    return c_out, h_out, ip_new, attribution, current_step + 1
